# Initial kernel scaffold; baseline (speedup 1.0000x reference)
import jax
import jax.numpy as jnp
from jax.experimental import pallas as pl


def kernel(x, edge_index, batch_index, attn_W, attn_b, topk_w, W0, b0, g0, be0, W1, b1, g1, be1, W2, b2, g2, be2, Wf, bf):
    raise NotImplementedError("write your pallas kernel here")


# speedup vs baseline: 1.5419x; 1.5419x over previous
import jax
import jax.numpy as jnp
from jax.experimental import pallas as pl

N_NODES = 100000
G_GRAPHS = 64
BLK = 5000
NB = N_NODES // BLK
F = 128


def _conv_k1(z_ref, kf_ref, w_ref, b_ref, r_ref, s1_ref, s2_ref):
    i = pl.program_id(0)
    z = z_ref[...]
    r = jnp.maximum(
        jnp.dot(z, w_ref[...], preferred_element_type=jnp.float32) + b_ref[0, :],
        0.0,
    )
    r_ref[...] = r
    rk = r * kf_ref[...]
    p1 = jnp.sum(rk, axis=0)
    p2 = jnp.sum(rk * r, axis=0)

    @pl.when(i == 0)
    def _():
        s1_ref[...] = jnp.zeros_like(s1_ref)
        s2_ref[...] = jnp.zeros_like(s2_ref)

    s1_ref[...] += jnp.broadcast_to(p1[None, :], s1_ref.shape)
    s2_ref[...] += jnp.broadcast_to(p2[None, :], s2_ref.shape)


def _conv_k2(r_ref, s1_ref, s2_ref, g_ref, be_ref, kinv_ref, o_ref):
    kinv = kinv_ref[0, :]
    m = s1_ref[0, :] * kinv
    v = s2_ref[0, :] * kinv - m * m
    inv = jax.lax.rsqrt(v + 1e-5)
    o_ref[...] = g_ref[0, :] * ((r_ref[...] - m[None, :]) * inv[None, :]) + be_ref[0, :]


def _pool_k(h_ref, m_ref, o_ref):
    i = pl.program_id(0)

    @pl.when(i == 0)
    def _():
        o_ref[...] = jnp.full_like(o_ref, -jnp.inf)

    h = h_ref[...]
    mk = m_ref[...]
    rows = []
    for g in range(G_GRAPHS):
        rows.append(jnp.max(jnp.where(mk[:, g:g + 1] > 0, h, -jnp.inf), axis=0))
    o_ref[...] = jnp.maximum(o_ref[...], jnp.stack(rows, axis=0))


def _final_k(p_ref, wf_ref, bf_ref, o_ref):
    o = jnp.dot(p_ref[...], wf_ref[...], preferred_element_type=jnp.float32) + bf_ref[0, :]
    col = jax.lax.broadcasted_iota(jnp.int32, o.shape, 1)
    o = jnp.where(col < 3, o, -jnp.inf)
    mx = jnp.max(o, axis=1, keepdims=True)
    lse = jnp.log(jnp.sum(jnp.exp(o - mx), axis=1)) [:, None] + mx
    o_ref[...] = o - lse


def _row8(v):
    return jnp.broadcast_to(v.reshape(1, -1), (8, v.shape[-1]))


def _conv_dense(z, kf128, W, b, g, be, kinv):
    f32 = jnp.float32
    r, s1, s2 = pl.pallas_call(
        _conv_k1,
        grid=(NB,),
        in_specs=[
            pl.BlockSpec((BLK, F), lambda i: (i, 0)),
            pl.BlockSpec((BLK, F), lambda i: (i, 0)),
            pl.BlockSpec((F, F), lambda i: (0, 0)),
            pl.BlockSpec((8, F), lambda i: (0, 0)),
        ],
        out_specs=[
            pl.BlockSpec((BLK, F), lambda i: (i, 0)),
            pl.BlockSpec((8, F), lambda i: (0, 0)),
            pl.BlockSpec((8, F), lambda i: (0, 0)),
        ],
        out_shape=[
            jax.ShapeDtypeStruct((N_NODES, F), f32),
            jax.ShapeDtypeStruct((8, F), f32),
            jax.ShapeDtypeStruct((8, F), f32),
        ],
    )(z, kf128, W, _row8(b))
    h = pl.pallas_call(
        _conv_k2,
        grid=(NB,),
        in_specs=[
            pl.BlockSpec((BLK, F), lambda i: (i, 0)),
            pl.BlockSpec((8, F), lambda i: (0, 0)),
            pl.BlockSpec((8, F), lambda i: (0, 0)),
            pl.BlockSpec((8, F), lambda i: (0, 0)),
            pl.BlockSpec((8, F), lambda i: (0, 0)),
            pl.BlockSpec((8, F), lambda i: (0, 0)),
        ],
        out_specs=pl.BlockSpec((BLK, F), lambda i: (i, 0)),
        out_shape=jax.ShapeDtypeStruct((N_NODES, F), f32),
    )(r, s1, s2, _row8(g), _row8(be), jnp.broadcast_to(kinv, (8, F)))
    return h


def kernel(x, edge_index, batch_index, attn_W, attn_b, topk_w, W0, b0, g0, be0,
           W1, b1, g1, be1, W2, b2, g2, be2, Wf, bf):
    f32 = jnp.float32
    n = N_NODES
    src, dst = edge_index[0], edge_index[1]

    deg = jnp.zeros((n,), f32).at[dst].add(1.0, mode='drop') + 1.0
    dinv = jax.lax.rsqrt(deg)
    pa = (x @ attn_W)[:, 0]
    t = dinv * pa
    agg = jnp.zeros((n,), f32).at[dst].add(t[src], mode='drop')
    raw = (dinv * (agg + dinv * pa) + attn_b[0]) * topk_w[0]

    m = jax.ops.segment_max(raw, batch_index, num_segments=G_GRAPHS)
    e = jnp.exp(raw - m[batch_index])
    zs = jax.ops.segment_sum(e, batch_index, num_segments=G_GRAPHS)
    score = e / zs[batch_index]
    smax = jax.ops.segment_max(score, batch_index, num_segments=G_GRAPHS)
    thresh = jnp.minimum(smax[batch_index] - 1e-7, 0.1)
    keep = score > thresh
    keepf = keep.astype(f32)
    k = keepf.sum()
    kinv = 1.0 / k
    xp = x * jnp.where(keep, score, 0.0)[:, None]

    edge_keep = keep[src] & keep[dst]
    oob = jnp.asarray(n, dtype=src.dtype)
    esrc = jnp.where(edge_keep, src, oob)
    edst = jnp.where(edge_keep, dst, oob)
    deg2 = jnp.zeros((n,), f32).at[edst].add(1.0, mode='drop') + 1.0
    dinv2 = jax.lax.rsqrt(deg2)
    d2c = dinv2[:, None]

    kf128 = jnp.broadcast_to(keepf[:, None], (n, F))

    h = xp
    for W, b, g, be in ((W0, b0, g0, be0), (W1, b1, g1, be1), (W2, b2, g2, be2)):
        t = d2c * h
        agg = jnp.zeros((n, h.shape[1]), f32).at[edst].add(t[esrc], mode='drop')
        z = d2c * agg + (d2c * d2c) * h
        zp = jnp.pad(z, ((0, 0), (0, F - z.shape[1]))) if z.shape[1] != F else z
        Wp = jnp.pad(W, ((0, F - W.shape[0]), (0, 0))) if W.shape[0] != F else W
        h = _conv_dense(zp, kf128, Wp, b, g, be, kinv)

    Mk = jax.nn.one_hot(batch_index, F, dtype=f32) * keepf[:, None]
    pooled = pl.pallas_call(
        _pool_k,
        grid=(NB,),
        in_specs=[
            pl.BlockSpec((BLK, F), lambda i: (i, 0)),
            pl.BlockSpec((BLK, F), lambda i: (i, 0)),
        ],
        out_specs=pl.BlockSpec((G_GRAPHS, F), lambda i: (0, 0)),
        out_shape=jax.ShapeDtypeStruct((G_GRAPHS, F), f32),
    )(h, Mk)

    Wfp = jnp.pad(Wf, ((0, 0), (0, F - Wf.shape[1])))
    bfp = jnp.pad(bf, (0, F - bf.shape[0]))
    out = pl.pallas_call(
        _final_k,
        in_specs=[
            pl.BlockSpec((G_GRAPHS, F), lambda: (0, 0)),
            pl.BlockSpec((F, F), lambda: (0, 0)),
            pl.BlockSpec((8, F), lambda: (0, 0)),
        ],
        out_specs=pl.BlockSpec((G_GRAPHS, F), lambda: (0, 0)),
        out_shape=jax.ShapeDtypeStruct((G_GRAPHS, F), f32),
    )(pooled, Wfp, _row8(bfp))
    return out[:, :3]
